# baseline (speedup 1.0000x reference)
import functools

import jax
import jax.numpy as jnp
from jax import lax
from jax.experimental import pallas as pl
from jax.experimental.pallas import tpu as pltpu
from jax.experimental.pallas import tpu_sc as plsc

BATCH = 64
SEQ = 2048
D = 512
VOCAB = 52
N_TOK = BATCH * SEQ
NC = 2
NS = 16
NW = NC * NS
TOK_PER_W = N_TOK // NW
CHUNK = 64
N_CHUNKS = TOK_PER_W // CHUNK
L = 16
GRP = CHUNK // L


@functools.partial(
    pl.kernel,
    mesh=plsc.VectorSubcoreMesh(core_axis_name="c", subcore_axis_name="s"),
    out_type=jax.ShapeDtypeStruct((N_TOK * D,), jnp.float32),
    scratch_types=[
        pltpu.VMEM((VOCAB * D,), jnp.float32),
        pltpu.VMEM((TOK_PER_W,), jnp.int32),
        pltpu.VMEM((CHUNK * D,), jnp.float32),
        pltpu.VMEM((CHUNK * D,), jnp.float32),
        pltpu.SemaphoreType.DMA,
        pltpu.SemaphoreType.DMA,
    ],
    compiler_params=pltpu.CompilerParams(
        use_tc_tiling_on_sc=False, needs_layout_passes=False),
)
def _embed_lookup(tokens_hbm, table_hbm, out_hbm, table_v, idx_v,
                  buf0, buf1, ss0, ss1):
    wid = lax.axis_index("s") * NC + lax.axis_index("c")
    base = wid * TOK_PER_W
    pltpu.sync_copy(table_hbm, table_v)
    pltpu.sync_copy(tokens_hbm.at[pl.ds(base, TOK_PER_W)], idx_v)

    lanes = lax.broadcasted_iota(jnp.int32, (L,), 0)
    posb = [(lanes + g * L) * D for g in range(GRP)]

    def fill(c, buf):
        tokb = [idx_v[pl.ds(c * CHUNK + g * L, L)] * D for g in range(GRP)]

        def dbody(d, carry):
            dv = jnp.full((L,), d, jnp.int32)
            for g in range(GRP):
                vals = plsc.load_gather(table_v, [tokb[g] + dv])
                plsc.store_scatter(buf, [posb[g] + dv], vals)
            return carry

        lax.fori_loop(0, D, dbody, 0, unroll=8)

    def s_start(c, buf, sem):
        pltpu.async_copy(
            buf, out_hbm.at[pl.ds((base + c * CHUNK) * D, CHUNK * D)], sem)

    def s_wait(buf, sem):
        pltpu.make_async_copy(
            buf, out_hbm.at[pl.ds(base * D, CHUNK * D)], sem).wait()

    fill(0, buf0)
    s_start(0, buf0, ss0)
    fill(1, buf1)
    s_start(1, buf1, ss1)

    def body(g, carry):
        c0 = 2 * g + 2
        s_wait(buf0, ss0)
        fill(c0, buf0)
        s_start(c0, buf0, ss0)
        s_wait(buf1, ss1)
        fill(c0 + 1, buf1)
        s_start(c0 + 1, buf1, ss1)
        return carry

    lax.fori_loop(0, N_CHUNKS // 2 - 1, body, 0)
    s_wait(buf0, ss0)
    s_wait(buf1, ss1)


def kernel(tokens, embed_weight):
    flat = tokens.reshape(-1).astype(jnp.int32)
    out = _embed_lookup(flat, embed_weight.reshape(-1))
    return out.reshape(BATCH, SEQ, D)


# speedup vs baseline: 2.2510x; 2.2510x over previous
import functools

import jax
import jax.numpy as jnp
from jax import lax
from jax.experimental import pallas as pl
from jax.experimental.pallas import tpu as pltpu
from jax.experimental.pallas import tpu_sc as plsc

BATCH = 64
SEQ = 2048
D = 512
VOCAB = 52
N_TOK = BATCH * SEQ
NC = 2
NS = 16
NW = NC * NS
TOK_PER_W = N_TOK // NW
CHUNK = 64
N_CHUNKS = TOK_PER_W // CHUNK
L = 16
GRP = CHUNK // L


@functools.partial(
    pl.kernel,
    mesh=plsc.VectorSubcoreMesh(core_axis_name="c", subcore_axis_name="s"),
    out_type=jax.ShapeDtypeStruct((N_TOK * D,), jnp.float32),
    scratch_types=[
        pltpu.VMEM((VOCAB * D,), jnp.float32),
        pltpu.VMEM((TOK_PER_W,), jnp.int32),
        pltpu.VMEM((CHUNK * D,), jnp.float32),
        pltpu.VMEM((CHUNK * D,), jnp.float32),
        pltpu.SemaphoreType.DMA,
        pltpu.SemaphoreType.DMA,
    ],
    compiler_params=pltpu.CompilerParams(
        use_tc_tiling_on_sc=False, needs_layout_passes=False),
)
def _embed_lookup(tokens_hbm, table_hbm, out_hbm, table_v, idx_v,
                  buf0, buf1, ss0, ss1):
    wid = lax.axis_index("s") * NC + lax.axis_index("c")
    base = wid * TOK_PER_W
    pltpu.sync_copy(table_hbm, table_v)
    pltpu.sync_copy(tokens_hbm.at[pl.ds(base, TOK_PER_W)], idx_v)

    lanes = lax.broadcasted_iota(jnp.int32, (L,), 0)
    posb = [(lanes + g * L) * D for g in range(GRP)]

    def fill(c, buf):
        tokb = [idx_v[pl.ds(c * CHUNK + g * L, L)] * D for g in range(GRP)]

        @plsc.parallel_loop(0, D, unroll=8)
        def dbody(d):
            dv = jnp.full((L,), d, jnp.int32)
            for g in range(GRP):
                vals = plsc.load_gather(table_v, [tokb[g] + dv])
                plsc.store_scatter(buf, [posb[g] + dv], vals)

    def s_start(c, buf, sem):
        pltpu.async_copy(
            buf, out_hbm.at[pl.ds((base + c * CHUNK) * D, CHUNK * D)], sem)

    def s_wait(buf, sem):
        pltpu.make_async_copy(
            buf, out_hbm.at[pl.ds(base * D, CHUNK * D)], sem).wait()

    fill(0, buf0)
    s_start(0, buf0, ss0)
    fill(1, buf1)
    s_start(1, buf1, ss1)

    def body(g, carry):
        c0 = 2 * g + 2
        s_wait(buf0, ss0)
        fill(c0, buf0)
        s_start(c0, buf0, ss0)
        s_wait(buf1, ss1)
        fill(c0 + 1, buf1)
        s_start(c0 + 1, buf1, ss1)
        return carry

    lax.fori_loop(0, N_CHUNKS // 2 - 1, body, 0)
    s_wait(buf0, ss0)
    s_wait(buf1, ss1)


def kernel(tokens, embed_weight):
    flat = tokens.reshape(-1).astype(jnp.int32)
    out = _embed_lookup(flat, embed_weight.reshape(-1))
    return out.reshape(BATCH, SEQ, D)


# speedup vs baseline: 6.9418x; 3.0838x over previous
import functools

import jax
import jax.numpy as jnp
from jax import lax
from jax.experimental import pallas as pl
from jax.experimental.pallas import tpu as pltpu
from jax.experimental.pallas import tpu_sc as plsc

BATCH = 64
SEQ = 2048
D = 512
VOCAB = 52
N_TOK = BATCH * SEQ
NC = 2
NS = 16
NW = NC * NS
TOK_PER_W = N_TOK // NW
CHUNK = 64
N_CHUNKS = TOK_PER_W // CHUNK
L = 16
GRP = CHUNK // L


@functools.partial(
    pl.kernel,
    mesh=plsc.VectorSubcoreMesh(core_axis_name="c", subcore_axis_name="s"),
    out_type=jax.ShapeDtypeStruct((N_TOK * D,), jnp.float32),
    scratch_types=[
        pltpu.VMEM((VOCAB * D,), jnp.float32),
        pltpu.VMEM((TOK_PER_W,), jnp.int32),
        pltpu.VMEM((CHUNK * D,), jnp.float32),
        pltpu.VMEM((CHUNK * D,), jnp.float32),
        pltpu.SemaphoreType.DMA,
        pltpu.SemaphoreType.DMA,
    ],
    compiler_params=pltpu.CompilerParams(
        use_tc_tiling_on_sc=False, needs_layout_passes=False),
)
def _embed_lookup(tokens_hbm, table_hbm, out_hbm, table_v, idx_v,
                  buf0, buf1, ss0, ss1):
    wid = lax.axis_index("s") * NC + lax.axis_index("c")
    base = wid * TOK_PER_W
    pltpu.sync_copy(table_hbm, table_v)
    pltpu.sync_copy(tokens_hbm.at[pl.ds(base, TOK_PER_W)], idx_v)

    def fill(c, buf):
        def gbody(g, carry):
            tok16 = idx_v[pl.ds(pl.multiple_of(c * CHUNK + g * L, L), L)]
            tb = [tok16[j] * D for j in range(L)]
            ob = [(g * L + j) * D for j in range(L)]

            @plsc.parallel_loop(0, D, step=L, unroll=2)
            def kbody(k):
                for j in range(L):
                    buf[pl.ds(ob[j] + k, L)] = table_v[pl.ds(tb[j] + k, L)]

            return carry

        lax.fori_loop(0, GRP, gbody, 0)

    def s_start(c, buf, sem):
        pltpu.async_copy(
            buf, out_hbm.at[pl.ds((base + c * CHUNK) * D, CHUNK * D)], sem)

    def s_wait(buf, sem):
        pltpu.make_async_copy(
            buf, out_hbm.at[pl.ds(base * D, CHUNK * D)], sem).wait()

    fill(0, buf0)
    s_start(0, buf0, ss0)
    fill(1, buf1)
    s_start(1, buf1, ss1)

    def body(g, carry):
        c0 = 2 * g + 2
        s_wait(buf0, ss0)
        fill(c0, buf0)
        s_start(c0, buf0, ss0)
        s_wait(buf1, ss1)
        fill(c0 + 1, buf1)
        s_start(c0 + 1, buf1, ss1)
        return carry

    lax.fori_loop(0, N_CHUNKS // 2 - 1, body, 0)
    s_wait(buf0, ss0)
    s_wait(buf1, ss1)


def kernel(tokens, embed_weight):
    flat = tokens.reshape(-1).astype(jnp.int32)
    out = _embed_lookup(flat, embed_weight.reshape(-1))
    return out.reshape(BATCH, SEQ, D)


# speedup vs baseline: 6.9681x; 1.0038x over previous
import functools

import jax
import jax.numpy as jnp
from jax import lax
from jax.experimental import pallas as pl
from jax.experimental.pallas import tpu as pltpu
from jax.experimental.pallas import tpu_sc as plsc

BATCH = 64
SEQ = 2048
D = 512
VOCAB = 52
N_TOK = BATCH * SEQ
NC = 2
NS = 16
NW = NC * NS
TOK_PER_W = N_TOK // NW
CHUNK = 64
N_CHUNKS = TOK_PER_W // CHUNK
L = 16
GRP = CHUNK // L


@functools.partial(
    pl.kernel,
    mesh=plsc.VectorSubcoreMesh(core_axis_name="c", subcore_axis_name="s"),
    out_type=jax.ShapeDtypeStruct((N_TOK * D,), jnp.float32),
    scratch_types=[
        pltpu.VMEM((VOCAB * D,), jnp.float32),
        pltpu.VMEM((TOK_PER_W,), jnp.int32),
        pltpu.VMEM((CHUNK * D,), jnp.float32),
        pltpu.VMEM((CHUNK * D,), jnp.float32),
        pltpu.SemaphoreType.DMA,
        pltpu.SemaphoreType.DMA,
    ],
    compiler_params=pltpu.CompilerParams(
        use_tc_tiling_on_sc=False, needs_layout_passes=False),
)
def _embed_lookup(tokens_hbm, table_hbm, out_hbm, table_v, idx_v,
                  buf0, buf1, ss0, ss1):
    wid = lax.axis_index("s") * NC + lax.axis_index("c")
    base = wid * TOK_PER_W
    pltpu.sync_copy(table_hbm, table_v)
    pltpu.sync_copy(tokens_hbm.at[pl.ds(base, TOK_PER_W)], idx_v)

    def fill(c, buf):
        def gbody(g, carry):
            tok16 = idx_v[pl.ds(pl.multiple_of(c * CHUNK + g * L, L), L)]
            tb = [tok16[j] * D for j in range(L)]
            ob = [(g * L + j) * D for j in range(L)]

            @plsc.parallel_loop(0, D, step=L, unroll=8)
            def kbody(k):
                for j in range(L):
                    buf[pl.ds(ob[j] + k, L)] = table_v[pl.ds(tb[j] + k, L)]

            return carry

        lax.fori_loop(0, GRP, gbody, 0)

    def s_start(c, buf, sem):
        pltpu.async_copy(
            buf, out_hbm.at[pl.ds((base + c * CHUNK) * D, CHUNK * D)], sem)

    def s_wait(buf, sem):
        pltpu.make_async_copy(
            buf, out_hbm.at[pl.ds(base * D, CHUNK * D)], sem).wait()

    fill(0, buf0)
    s_start(0, buf0, ss0)
    fill(1, buf1)
    s_start(1, buf1, ss1)

    def body(g, carry):
        c0 = 2 * g + 2
        s_wait(buf0, ss0)
        fill(c0, buf0)
        s_start(c0, buf0, ss0)
        s_wait(buf1, ss1)
        fill(c0 + 1, buf1)
        s_start(c0 + 1, buf1, ss1)
        return carry

    lax.fori_loop(0, N_CHUNKS // 2 - 1, body, 0)
    s_wait(buf0, ss0)
    s_wait(buf1, ss1)


def kernel(tokens, embed_weight):
    flat = tokens.reshape(-1).astype(jnp.int32)
    out = _embed_lookup(flat, embed_weight.reshape(-1))
    return out.reshape(BATCH, SEQ, D)


# speedup vs baseline: 7.1500x; 1.0261x over previous
import functools

import jax
import jax.numpy as jnp
from jax import lax
from jax.experimental import pallas as pl
from jax.experimental.pallas import tpu as pltpu
from jax.experimental.pallas import tpu_sc as plsc

BATCH = 64
SEQ = 2048
D = 512
VOCAB = 52
N_TOK = BATCH * SEQ
NC = 2
NS = 16
NW = NC * NS
TOK_PER_W = N_TOK // NW
L = 16
N_GRP = TOK_PER_W // L


@functools.partial(
    pl.kernel,
    mesh=plsc.VectorSubcoreMesh(core_axis_name="c", subcore_axis_name="s"),
    out_type=jax.ShapeDtypeStruct((N_TOK * D,), jnp.float32),
    scratch_types=[
        pltpu.VMEM((VOCAB * D,), jnp.float32),
        pltpu.VMEM((TOK_PER_W,), jnp.int32),
        pltpu.SemaphoreType.DMA,
    ],
    compiler_params=pltpu.CompilerParams(
        use_tc_tiling_on_sc=False, needs_layout_passes=False),
)
def _embed_lookup(tokens_hbm, table_hbm, out_hbm, table_v, idx_v, sem):
    wid = lax.axis_index("s") * NC + lax.axis_index("c")
    base = wid * TOK_PER_W
    pltpu.sync_copy(table_hbm, table_v)
    pltpu.sync_copy(tokens_hbm.at[pl.ds(base, TOK_PER_W)], idx_v)

    def wait_one():
        pltpu.make_async_copy(
            table_v.at[pl.ds(0, D)],
            out_hbm.at[pl.ds(base * D, D)], sem).wait()

    def gbody(g, carry):
        tok16 = idx_v[pl.ds(pl.multiple_of(g * L, L), L)]
        for j in range(L):
            pltpu.async_copy(
                table_v.at[pl.ds(tok16[j] * D, D)],
                out_hbm.at[pl.ds((base + g * L + j) * D, D)], sem)

        @pl.when(g >= 1)
        def _drain_prev_group():
            for _ in range(L):
                wait_one()

        return carry

    lax.fori_loop(0, N_GRP, gbody, 0)
    for _ in range(L):
        wait_one()


def kernel(tokens, embed_weight):
    flat = tokens.reshape(-1).astype(jnp.int32)
    out = _embed_lookup(flat, embed_weight.reshape(-1))
    return out.reshape(BATCH, SEQ, D)


# speedup vs baseline: 7.9246x; 1.1083x over previous
import functools

import jax
import jax.numpy as jnp
from jax import lax
from jax.experimental import pallas as pl
from jax.experimental.pallas import tpu as pltpu
from jax.experimental.pallas import tpu_sc as plsc

BATCH = 64
SEQ = 2048
D = 512
VOCAB = 52
N_TOK = BATCH * SEQ
NC = 2
NS = 16
NW = NC * NS
L = 16

SC_FRAC_NUM = 8
SC_TOK = N_TOK * SC_FRAC_NUM // 32
TC_TOK = N_TOK - SC_TOK
TOK_PER_W = SC_TOK // NW
N_GRP = TOK_PER_W // L

BLK = 4096
NBLK = TC_TOK // BLK
VPAD = 64


@functools.partial(
    pl.kernel,
    mesh=plsc.VectorSubcoreMesh(core_axis_name="c", subcore_axis_name="s"),
    out_type=jax.ShapeDtypeStruct((SC_TOK * D,), jnp.float32),
    scratch_types=[
        pltpu.VMEM((VOCAB * D,), jnp.float32),
        pltpu.VMEM((TOK_PER_W,), jnp.int32),
        pltpu.SemaphoreType.DMA,
    ],
    compiler_params=pltpu.CompilerParams(
        use_tc_tiling_on_sc=False, needs_layout_passes=False),
)
def _embed_sc(tokens_hbm, table_hbm, out_hbm, table_v, idx_v, sem):
    wid = lax.axis_index("s") * NC + lax.axis_index("c")
    base = wid * TOK_PER_W
    pltpu.sync_copy(table_hbm, table_v)
    pltpu.sync_copy(tokens_hbm.at[pl.ds(base, TOK_PER_W)], idx_v)

    def wait_one():
        pltpu.make_async_copy(
            table_v.at[pl.ds(0, D)],
            out_hbm.at[pl.ds(base * D, D)], sem).wait()

    def gbody(g, carry):
        tok16 = idx_v[pl.ds(pl.multiple_of(g * L, L), L)]
        for j in range(L):
            pltpu.async_copy(
                table_v.at[pl.ds(tok16[j] * D, D)],
                out_hbm.at[pl.ds((base + g * L + j) * D, D)], sem)

        @pl.when(g >= 1)
        def _drain_prev_group():
            for _ in range(L):
                wait_one()

        return carry

    lax.fori_loop(0, N_GRP, gbody, 0)
    for _ in range(L):
        wait_one()


def _tc_body(tok_ref, tab_ref, out_ref):
    tok = tok_ref[0, 0, :].reshape(BLK, 1)
    iota = lax.broadcasted_iota(jnp.int32, (BLK, VPAD), 1)
    onehot = (tok == iota).astype(jnp.float32)
    out_ref[...] = jnp.dot(onehot, tab_ref[...],
                           preferred_element_type=jnp.float32)


def _embed_tc(tokens3, table_pad):
    return pl.pallas_call(
        _tc_body,
        grid=(NBLK,),
        in_specs=[
            pl.BlockSpec((1, 1, BLK), lambda i: (i, 0, 0)),
            pl.BlockSpec((VPAD, D), lambda i: (0, 0)),
        ],
        out_specs=pl.BlockSpec((BLK, D), lambda i: (i, 0)),
        out_shape=jax.ShapeDtypeStruct((TC_TOK, D), jnp.float32),
    )(tokens3, table_pad)


def kernel(tokens, embed_weight):
    flat = tokens.reshape(-1).astype(jnp.int32)
    sc_out = _embed_sc(flat[:SC_TOK], embed_weight.reshape(-1))
    tab = jnp.zeros((VPAD, D), jnp.float32).at[:VOCAB].set(embed_weight)
    tc_out = _embed_tc(flat[SC_TOK:].reshape(NBLK, 1, BLK), tab)
    out = jnp.concatenate([sc_out.reshape(SC_TOK, D), tc_out], axis=0)
    return out.reshape(BATCH, SEQ, D)
